# baseline (speedup 1.0000x reference)
import functools
import math

import jax
import jax.numpy as jnp
from jax import lax
from jax.experimental import pallas as pl
from jax.experimental.pallas import tpu as pltpu
from jax.experimental.pallas import tpu_sc as plsc

VOCAB = 100000
D = 128
B = 4096
L = 50

NC = 2
NS = 16
LANES = 16
NW = NC * NS

BW = B // NW
CHUNKS = BW
NB = 8
G = 3

SCALE = math.sqrt(D)

_mesh = plsc.VectorSubcoreMesh(core_axis_name="c", subcore_axis_name="s")

_scratch = (
    [pltpu.VMEM((BW, L), jnp.int32)]
    + [pltpu.VMEM((L, 1, D), jnp.float32) for _ in range(NB)]
    + [pltpu.SemaphoreType.DMA for _ in range(2 * NB)]
)


@functools.partial(
    pl.kernel,
    out_type=jax.ShapeDtypeStruct((L, B, D), jnp.float32),
    mesh=_mesh,
    scratch_types=_scratch,
    compiler_params=pltpu.CompilerParams(use_tc_tiling_on_sc=True),
)
def _emb_kernel(x_hbm, table_hbm, out_hbm, idx_v, *scratch):
    bufs = scratch[:NB]
    gsems = scratch[NB:2 * NB]
    osems = scratch[2 * NB:]

    wid = lax.axis_index("s") * NC + lax.axis_index("c")
    batch_base = wid * BW
    pltpu.sync_copy(x_hbm.at[pl.ds(batch_base, BW)], idx_v)

    def start_gather(c, s):
        pltpu.async_copy(table_hbm.at[idx_v.at[c]], bufs[s].at[:, 0], gsems[s])

    def wait_gather(c, s):
        pltpu.make_async_copy(
            table_hbm.at[idx_v.at[c]], bufs[s].at[:, 0], gsems[s]
        ).wait()

    def start_out(c, s):
        pltpu.async_copy(bufs[s], out_hbm.at[:, pl.ds(batch_base + c, 1)], osems[s])

    def wait_out(c, s):
        pltpu.make_async_copy(
            bufs[s], out_hbm.at[:, pl.ds(batch_base + c, 1)], osems[s]
        ).wait()

    def scale(s):
        buf = bufs[s]

        @plsc.parallel_loop(0, L, unroll=2)
        def _(i):
            for j in range(D // LANES):
                sl = pl.ds(j * LANES, LANES)
                buf[i, 0, sl] = buf[i, 0, sl] * SCALE

    def step(c, s, do_wait_out, do_gather):
        wait_gather(c, s)
        scale(s)
        start_out(c, s)
        if do_gather:
            s2 = (s + G) % NB
            if do_wait_out:
                wait_out(c + G - NB, s2)
            start_gather(c + G, s2)

    for c in range(G):
        start_gather(c, c)

    for c in range(NB):
        step(c, c, c >= NB - G, True)

    def outer(g, carry):
        for b in range(NB):
            c = g * NB + b
            step(c, b, True, True)
        return carry

    lax.fori_loop(1, CHUNKS // NB - 1, outer, 0)

    for c in range(CHUNKS - NB, CHUNKS):
        step(c, c % NB, True, c + G < CHUNKS)
    for c in range(CHUNKS - NB, CHUNKS):
        wait_out(c, c % NB)


def kernel(x, table):
    out = _emb_kernel(x.astype(jnp.int32), table)
    return out.transpose(1, 0, 2)


# speedup vs baseline: 1.0880x; 1.0880x over previous
import functools
import math

import jax
import jax.numpy as jnp
from jax import lax
from jax.experimental import pallas as pl
from jax.experimental.pallas import tpu as pltpu
from jax.experimental.pallas import tpu_sc as plsc

VOCAB = 100000
D = 128
B = 4096
L = 50

NC = 2
NS = 16
LANES = 16
NW = NC * NS

BW = B // NW
CHUNKS = BW
NB = 8

SCALE = math.sqrt(D)

_mesh = plsc.VectorSubcoreMesh(core_axis_name="c", subcore_axis_name="s")

_scratch = (
    [pltpu.VMEM((BW, L), jnp.int32)]
    + [pltpu.VMEM((L, 1, D), jnp.float32) for _ in range(NB)]
    + [pltpu.SemaphoreType.DMA for _ in range(2 * NB)]
)


@functools.partial(
    pl.kernel,
    out_type=jax.ShapeDtypeStruct((L, B, D), jnp.float32),
    mesh=_mesh,
    scratch_types=_scratch,
    compiler_params=pltpu.CompilerParams(use_tc_tiling_on_sc=True),
)
def _emb_kernel(x_hbm, table_hbm, out_hbm, idx_v, *scratch):
    bufs = scratch[:NB]
    gsems = scratch[NB:2 * NB]
    osems = scratch[2 * NB:]

    wid = lax.axis_index("s") * NC + lax.axis_index("c")
    batch_base = wid * BW
    pltpu.sync_copy(x_hbm.at[pl.ds(batch_base, BW)], idx_v)

    def start_gather(c, s):
        pltpu.async_copy(table_hbm.at[idx_v.at[c]], bufs[s].at[:, 0], gsems[s])

    def wait_gather(c, s):
        pltpu.make_async_copy(
            table_hbm.at[idx_v.at[c]], bufs[s].at[:, 0], gsems[s]
        ).wait()

    def start_out(c, s):
        pltpu.async_copy(bufs[s], out_hbm.at[:, pl.ds(batch_base + c, 1)], osems[s])

    def wait_out(c, s):
        pltpu.make_async_copy(
            bufs[s], out_hbm.at[:, pl.ds(batch_base + c, 1)], osems[s]
        ).wait()

    def scale(s):
        buf = bufs[s]

        @plsc.parallel_loop(0, L, unroll=2)
        def _(i):
            for j in range(D // LANES):
                sl = pl.ds(j * LANES, LANES)
                buf[i, 0, sl] = buf[i, 0, sl] * SCALE

    for c in range(NB - 1):
        start_gather(c, c)
    wait_gather(0, 0)
    scale(0)
    start_out(0, 0)
    start_gather(NB - 1, NB - 1)

    def outer(g, carry):
        for b in range(NB):
            c = 1 + g * NB + b
            s = (1 + b) % NB
            wait_gather(c, s)
            scale(s)
            start_out(c, s)
            s2 = (s + NB - 1) % NB
            wait_out(c - 1, s2)
            start_gather(c + NB - 1, s2)
        return carry

    lax.fori_loop(0, (CHUNKS - NB) // NB, outer, 0)

    for c in range(CHUNKS - NB + 1, CHUNKS):
        s = c % NB
        wait_gather(c, s)
        scale(s)
        start_out(c, s)
    for c in range(CHUNKS - NB, CHUNKS):
        wait_out(c, c % NB)


def kernel(x, table):
    out = _emb_kernel(x.astype(jnp.int32), table)
    return out.transpose(1, 0, 2)
